# baseline (speedup 1.0000x reference)
import functools

import jax
import jax.numpy as jnp
from jax import lax
from jax.experimental import pallas as pl
from jax.experimental.pallas import tpu as pltpu
from jax.experimental.pallas import tpu_sc as plsc

N = 1_000_000
C = 10
B = 15
THRESHOLD = 0.01

NW = 32
WS = 31_248
TAIL = N - NW * WS
CS = 1_008
NCHUNK = WS // CS
ROW = 160
ROWP = 161
CAT = 16 * ROWP
L = 16
EU = 5
LU = 7


def _bin_many(ps):
    one = jnp.float32(1.0)
    delta = one / jnp.float32(15.0)
    thr = jnp.float32(THRESHOLD)
    i0 = [(p * jnp.float32(15.0)).astype(jnp.int32) for p in ps]
    f = [x.astype(jnp.float32) for x in i0]
    lo = [x * delta for x in f]
    hi = [(x + one) * delta for x in f]
    down = [jnp.where(p <= l, jnp.int32(1), jnp.int32(0))
            for p, l in zip(ps, lo)]
    up = [jnp.where(p >= h, jnp.int32(1), jnp.int32(0))
          for p, h in zip(ps, hi)]
    c1 = [a - d + u for a, d, u in zip(i0, down, up)]
    valid = [(p != l) & (p != h) & (p > thr)
             for p, l, h in zip(ps, lo, hi)]
    return c1, valid


def _sc_partials(probas_flat, labels, base, ws, cs, nch, tail_base, tail_n,
                 tail_lu, mlu):
    mesh = plsc.VectorSubcoreMesh(core_axis_name="c", subcore_axis_name="s")

    @functools.partial(
        pl.kernel,
        mesh=mesh,
        out_type=jax.ShapeDtypeStruct((NW, 3 * ROW), jnp.float32),
        compiler_params=pltpu.CompilerParams(needs_layout_passes=False),
        scratch_types=[
            pltpu.VMEM((cs * C,), jnp.float32),
            pltpu.VMEM((cs * C,), jnp.float32),
            pltpu.VMEM((cs,), jnp.int32),
            pltpu.VMEM((cs,), jnp.int32),
            pltpu.VMEM((3 * CAT,), jnp.float32),
            pltpu.VMEM((3 * ROW,), jnp.float32),
            pltpu.SemaphoreType.DMA,
            pltpu.SemaphoreType.DMA,
        ],
    )
    def k(p_hbm, l_hbm, out_hbm, pbufa, pbufb, lbufa, lbufb, tab, stage,
          sema, semb):
        wid = lax.axis_index("s") * 2 + lax.axis_index("c")
        ones = jnp.full((L,), 1.0, jnp.float32)
        iota = lax.iota(jnp.int32, L)
        lane_base = iota * ROWP
        lane10 = iota * C
        acc_base = lane_base + 2 * CAT
        cnt_bases = []
        conf_bases = []
        for ui in range(EU):
            cls = lax.rem(iota + L * ui, jnp.int32(C))
            cnt_bases.append(lane_base + cls * B)
            conf_bases.append(lane_base + cls * B + CAT)

        def zero_body(i, _):
            tab[pl.ds(i * L, L)] = jnp.zeros((L,), jnp.float32)
            return _
        lax.fori_loop(0, 3 * CAT // L, zero_body, None)

        def issue(c, pbuf, lbuf, sem):
            s_off = base + wid * ws + c * cs
            pltpu.async_copy(p_hbm.at[pl.ds(s_off * C, cs * C)], pbuf, sem)
            pltpu.async_copy(l_hbm.at[pl.ds(s_off, cs)], lbuf, sem)

        def drain(pbuf, lbuf, sem):
            pltpu.make_async_copy(p_hbm.at[pl.ds(0, cs * C)], pbuf,
                                  sem).wait()
            pltpu.make_async_copy(l_hbm.at[pl.ds(0, cs)], lbuf, sem).wait()

        def compute(pbuf, lbuf, n_samp, lu):
            @plsc.parallel_loop(0, n_samp * C // (EU * L))
            def elem_body(vo):
                base = vo * (EU * L)
                ps = [pbuf[pl.ds(base + ui * L, L)] for ui in range(EU)]
                c1s, valids = _bin_many(ps)
                for ui in range(EU):
                    plsc.addupdate_scatter(tab, [cnt_bases[ui] + c1s[ui]],
                                           ones, mask=valids[ui])
                    plsc.addupdate_scatter(tab, [conf_bases[ui] + c1s[ui]],
                                           ps[ui], mask=valids[ui])

            @plsc.parallel_loop(0, n_samp // (lu * L))
            def lbl_body(uo):
                offs = [uo * (lu * L) + li * L for li in range(lu)]
                lbls = [lbuf[pl.ds(off, L)] for off in offs]
                pvs = [plsc.load_gather(pbuf, [off * C + lane10 + lbl])
                       for off, lbl in zip(offs, lbls)]
                c1s, valids = _bin_many(pvs)
                for li in range(lu):
                    si = acc_base + lbls[li] * B + c1s[li]
                    plsc.addupdate_scatter(tab, [si], ones, mask=valids[li])

        issue(0, pbufa, lbufa, sema)

        def pair_body(i, _):
            issue(2 * i + 1, pbufb, lbufb, semb)
            drain(pbufa, lbufa, sema)
            compute(pbufa, lbufa, cs, mlu)
            issue(2 * i + 2, pbufa, lbufa, sema)
            drain(pbufb, lbufb, semb)
            compute(pbufb, lbufb, cs, mlu)
            return _

        lax.fori_loop(0, (nch - 1) // 2, pair_body, None)
        drain(pbufa, lbufa, sema)
        compute(pbufa, lbufa, cs, mlu)

        @pl.when(wid == 0)
        def _tail():
            pltpu.sync_copy(p_hbm.at[pl.ds(tail_base * C, tail_n * C)],
                            pbufa.at[pl.ds(0, tail_n * C)])
            pltpu.sync_copy(l_hbm.at[pl.ds(tail_base, tail_n)],
                            lbufa.at[pl.ds(0, tail_n)])
            compute(pbufa, lbufa, tail_n, tail_lu)

        def red_body(m, _):
            for cat in range(3):
                acc = jnp.zeros((L,), jnp.float32)
                for lane in range(L):
                    idx = iota + (cat * CAT + lane * ROWP + m * L)
                    acc = acc + plsc.load_gather(tab, [idx])
                stage[pl.ds(cat * ROW + m * L, L)] = acc
            return _
        lax.fori_loop(0, ROW // L, red_body, None)

        pltpu.sync_copy(stage, out_hbm.at[wid])

    return k(probas_flat, labels)


def kernel(probas, labels):
    lab = labels.astype(jnp.int32)
    half = N // 2
    hws, hcs = 15_600, 1_040
    parts = []
    for h in range(2):
        pf = jnp.maximum(probas[h * half:(h + 1) * half],
                         jnp.float32(-1.0)).reshape(-1)
        parts.append(_sc_partials(pf, lab[h * half:(h + 1) * half],
                                  0, hws, hcs, 15, NW * hws, 800, 5, 5))
    tot = (parts[0].sum(axis=0) + parts[1].sum(axis=0))
    tot = tot.reshape(3, ROW)[:, : C * B].reshape(3, C, B)
    cnt, conf_sum, acc_sum = tot[0], tot[1], tot[2]
    return jnp.stack([conf_sum / cnt, acc_sum / cnt, cnt], axis=0)


# speedup vs baseline: 1.1681x; 1.1681x over previous
import functools

import jax
import jax.numpy as jnp
from jax import lax
from jax.experimental import pallas as pl
from jax.experimental.pallas import tpu as pltpu
from jax.experimental.pallas import tpu_sc as plsc

N = 1_000_000
C = 10
B = 15
THRESHOLD = 0.01

NW = 32
WS = 31_248
TAIL = N - NW * WS
CS = 1_008
NCHUNK = WS // CS
ROW = 160
ROWP = 161
CAT = 16 * ROWP
L = 16
EU = 5
LU = 7


def _bin_many(ps):
    one = jnp.float32(1.0)
    delta = one / jnp.float32(15.0)
    thr = jnp.float32(THRESHOLD)
    i0 = [(p * jnp.float32(15.0)).astype(jnp.int32) for p in ps]
    f = [x.astype(jnp.float32) for x in i0]
    lo = [x * delta for x in f]
    hi = [(x + one) * delta for x in f]
    down = [jnp.where(p <= l, jnp.int32(1), jnp.int32(0))
            for p, l in zip(ps, lo)]
    up = [jnp.where(p >= h, jnp.int32(1), jnp.int32(0))
          for p, h in zip(ps, hi)]
    c1 = [a - d + u for a, d, u in zip(i0, down, up)]
    valid = [(p != l) & (p != h) & (p > thr)
             for p, l, h in zip(ps, lo, hi)]
    return c1, valid


def _sc_partials(probas_flat, labels, base, ws, cs, nch, tail_base, tail_n,
                 tail_lu, mlu):
    mesh = plsc.VectorSubcoreMesh(core_axis_name="c", subcore_axis_name="s")

    @functools.partial(
        pl.kernel,
        mesh=mesh,
        out_type=jax.ShapeDtypeStruct((NW, 3 * ROW), jnp.float32),
        compiler_params=pltpu.CompilerParams(needs_layout_passes=False),
        scratch_types=[
            pltpu.VMEM((cs * C,), jnp.float32),
            pltpu.VMEM((cs * C,), jnp.float32),
            pltpu.VMEM((cs,), jnp.int32),
            pltpu.VMEM((cs,), jnp.int32),
            pltpu.VMEM((3 * CAT,), jnp.float32),
            pltpu.VMEM((3 * ROW,), jnp.float32),
            pltpu.SemaphoreType.DMA,
            pltpu.SemaphoreType.DMA,
        ],
    )
    def k(p_hbm, l_hbm, out_hbm, pbufa, pbufb, lbufa, lbufb, tab, stage,
          sema, semb):
        wid = lax.axis_index("s") * 2 + lax.axis_index("c")
        ones = jnp.full((L,), 1.0, jnp.float32)
        iota = lax.iota(jnp.int32, L)
        lane_base = iota * ROWP
        lane10 = iota * C
        acc_base = lane_base + 2 * CAT
        cnt_bases = []
        conf_bases = []
        for ui in range(EU):
            cls = lax.rem(iota + L * ui, jnp.int32(C))
            cnt_bases.append(lane_base + cls * B)
            conf_bases.append(lane_base + cls * B + CAT)

        def zero_body(i, _):
            tab[pl.ds(i * L, L)] = jnp.zeros((L,), jnp.float32)
            return _
        lax.fori_loop(0, 3 * CAT // L, zero_body, None)

        def issue(c, pbuf, lbuf, sem):
            s_off = base + wid * ws + c * cs
            pltpu.async_copy(p_hbm.at[pl.ds(s_off * C, cs * C)], pbuf, sem)
            pltpu.async_copy(l_hbm.at[pl.ds(s_off, cs)], lbuf, sem)

        def drain(pbuf, lbuf, sem):
            pltpu.make_async_copy(p_hbm.at[pl.ds(0, cs * C)], pbuf,
                                  sem).wait()
            pltpu.make_async_copy(l_hbm.at[pl.ds(0, cs)], lbuf, sem).wait()

        def compute(pbuf, lbuf, n_samp, lu):
            @plsc.parallel_loop(0, n_samp * C // (EU * L))
            def elem_body(vo):
                base = vo * (EU * L)
                ps = [pbuf[pl.ds(base + ui * L, L)] for ui in range(EU)]
                c1s, valids = _bin_many(ps)
                for ui in range(EU):
                    plsc.addupdate_scatter(tab, [cnt_bases[ui] + c1s[ui]],
                                           ones, mask=valids[ui])
                    plsc.addupdate_scatter(tab, [conf_bases[ui] + c1s[ui]],
                                           ps[ui], mask=valids[ui])

            @plsc.parallel_loop(0, n_samp // (lu * L))
            def lbl_body(uo):
                offs = [uo * (lu * L) + li * L for li in range(lu)]
                lbls = [lbuf[pl.ds(off, L)] for off in offs]
                pvs = [plsc.load_gather(pbuf, [off * C + lane10 + lbl])
                       for off, lbl in zip(offs, lbls)]
                c1s, valids = _bin_many(pvs)
                for li in range(lu):
                    si = acc_base + lbls[li] * B + c1s[li]
                    plsc.addupdate_scatter(tab, [si], ones, mask=valids[li])

        issue(0, pbufa, lbufa, sema)

        def pair_body(i, _):
            issue(2 * i + 1, pbufb, lbufb, semb)
            drain(pbufa, lbufa, sema)
            compute(pbufa, lbufa, cs, mlu)
            issue(2 * i + 2, pbufa, lbufa, sema)
            drain(pbufb, lbufb, semb)
            compute(pbufb, lbufb, cs, mlu)
            return _

        lax.fori_loop(0, (nch - 1) // 2, pair_body, None)
        drain(pbufa, lbufa, sema)
        compute(pbufa, lbufa, cs, mlu)

        @pl.when(wid == 0)
        def _tail():
            pltpu.sync_copy(p_hbm.at[pl.ds(tail_base * C, tail_n * C)],
                            pbufa.at[pl.ds(0, tail_n * C)])
            pltpu.sync_copy(l_hbm.at[pl.ds(tail_base, tail_n)],
                            lbufa.at[pl.ds(0, tail_n)])
            compute(pbufa, lbufa, tail_n, tail_lu)

        def red_body(m, _):
            for cat in range(3):
                acc = jnp.zeros((L,), jnp.float32)
                for lane in range(L):
                    idx = iota + (cat * CAT + lane * ROWP + m * L)
                    acc = acc + plsc.load_gather(tab, [idx])
                stage[pl.ds(cat * ROW + m * L, L)] = acc
            return _
        lax.fori_loop(0, ROW // L, red_body, None)

        pltpu.sync_copy(stage, out_hbm.at[wid])

    return k(probas_flat, labels)


def kernel(probas, labels):
    partial = _sc_partials(probas.reshape(-1), labels.astype(jnp.int32),
                           0, WS, CS, NCHUNK, NW * WS, TAIL, 4, LU)
    tot = partial.sum(axis=0).reshape(3, ROW)[:, : C * B].reshape(3, C, B)
    cnt, conf_sum, acc_sum = tot[0], tot[1], tot[2]
    return jnp.stack([conf_sum / cnt, acc_sum / cnt, cnt], axis=0)
